# Initial kernel scaffold; baseline (speedup 1.0000x reference)
import jax
import jax.numpy as jnp
from jax.experimental import pallas as pl


def kernel(feat, edge_index):
    raise NotImplementedError("write your pallas kernel here")


# speedup vs baseline: 4.9177x; 4.9177x over previous
import functools

import jax
import jax.numpy as jnp
from jax import lax
from jax.experimental import pallas as pl
from jax.experimental.pallas import tpu as pltpu
from jax.experimental.pallas import tpu_sc as plsc

N_NODES = 10000
N_PAD = 10240
D_FEAT = 128
DH = 64
N_EDGES = 320000
CHUNK = 128
NCHUNK = 160
E_PAD = 16 * NCHUNK * CHUNK
NBUF = 2
TRASH_ROW = N_NODES
ROWS_PER_TILE = N_PAD // 16


def _make_sc_call():
  mesh = plsc.VectorSubcoreMesh(core_axis_name="c", subcore_axis_name="s")

  @functools.partial(
      pl.kernel,
      mesh=mesh,
      out_type=jax.ShapeDtypeStruct((2 * N_PAD, DH), jnp.float32),
      compiler_params=pltpu.CompilerParams(use_tc_tiling_on_sc=False),
      scratch_types=[
          pltpu.VMEM((NCHUNK, CHUNK), jnp.int32),
          pltpu.VMEM((NCHUNK, CHUNK), jnp.int32),
          pltpu.VMEM((NBUF, CHUNK, DH), jnp.float32),
          pltpu.VMEM_SHARED((N_PAD, DH), jnp.float32),
          pltpu.SemaphoreType.DMA,
          pltpu.SemaphoreType.DMA,
      ],
  )
  def sc_kernel(feat_hbm, src_hbm, dst_hbm, out_hbm,
                src_v, dst_v, rows_v, acc, sem0, sem1):
    c = lax.axis_index("c")
    s = lax.axis_index("s")
    sems = [sem0, sem1]

    zero16 = jnp.zeros((16,), jnp.float32)

    def _zrow(i, carry):
      for t in range(DH // 16):
        rows_v[0, i, pl.ds(t * 16, 16)] = zero16
      return carry

    lax.fori_loop(0, CHUNK, _zrow, 0)
    row0 = s * ROWS_PER_TILE
    for b in range(ROWS_PER_TILE // CHUNK):
      pltpu.sync_copy(rows_v.at[0], acc.at[pl.ds(row0 + b * CHUNK, CHUNK)])
    plsc.subcore_barrier()

    pltpu.sync_copy(src_hbm.at[c, s], src_v)
    pltpu.sync_copy(dst_hbm.at[s], dst_v)

    def _start_gather(j, b):
      pltpu.async_copy(feat_hbm.at[src_v.at[j]], rows_v.at[b], sems[b])

    def _wait_gather(j, b):
      pltpu.make_async_copy(
          feat_hbm.at[src_v.at[j]], rows_v.at[b], sems[b]).wait()

    for b in range(NBUF):
      _start_gather(b, b)

    def _group(g, carry):
      for b in range(NBUF):
        j = g * NBUF + b
        _wait_gather(j, b)
        pltpu.sync_copy(rows_v.at[b], acc.at[dst_v.at[j]], add=True)
        jn = j + NBUF

        @pl.when(jn < NCHUNK)
        def _():
          _start_gather(jn, b)

      return carry

    lax.fori_loop(0, NCHUNK // NBUF, _group, 0)
    plsc.subcore_barrier()

    pltpu.sync_copy(
        acc.at[pl.ds(row0, ROWS_PER_TILE)],
        out_hbm.at[pl.ds(c * N_PAD + row0, ROWS_PER_TILE)])

  return sc_kernel


_sc_call = _make_sc_call()


def kernel(feat, edge_index):
  ei = edge_index.astype(jnp.int32)
  npad = E_PAD - N_EDGES
  src = jnp.concatenate([ei[0], jnp.zeros((npad,), jnp.int32)])
  dst = jnp.concatenate([ei[1], jnp.full((npad,), TRASH_ROW, jnp.int32)])
  src2 = jnp.stack([src, src + N_NODES]).reshape(2, 16, NCHUNK, CHUNK)
  dst2 = dst.reshape(16, NCHUNK, CHUNK)
  feat2 = feat.reshape(N_NODES, 2, DH).transpose(1, 0, 2).reshape(2 * N_NODES, DH)
  out2 = _sc_call(feat2, src2, dst2)
  out2 = out2.reshape(2, N_PAD, DH)[:, :N_NODES, :]
  return out2.transpose(1, 0, 2).reshape(N_NODES, D_FEAT)


# speedup vs baseline: 4.9315x; 1.0028x over previous
import functools

import jax
import jax.numpy as jnp
from jax import lax
from jax.experimental import pallas as pl
from jax.experimental.pallas import tpu as pltpu
from jax.experimental.pallas import tpu_sc as plsc

N_NODES = 10000
N_PAD = 10240
D_FEAT = 128
DH = 64
N_EDGES = 320000
CHUNK = 128
NCHUNK = 160
E_PAD = 16 * NCHUNK * CHUNK
NBUF = 4
LAG = 2
TRASH_ROW = N_NODES
ROWS_PER_TILE = N_PAD // 16


def _make_sc_call():
  mesh = plsc.VectorSubcoreMesh(core_axis_name="c", subcore_axis_name="s")

  @functools.partial(
      pl.kernel,
      mesh=mesh,
      out_type=jax.ShapeDtypeStruct((2 * N_PAD, DH), jnp.float32),
      compiler_params=pltpu.CompilerParams(use_tc_tiling_on_sc=False),
      scratch_types=[
          pltpu.VMEM((NCHUNK, CHUNK), jnp.int32),
          pltpu.VMEM((NCHUNK, CHUNK), jnp.int32),
          pltpu.VMEM((NBUF, CHUNK, DH), jnp.float32),
          pltpu.VMEM_SHARED((N_PAD, DH), jnp.float32),
          pltpu.SemaphoreType.DMA((NBUF,)),
          pltpu.SemaphoreType.DMA((NBUF,)),
      ],
  )
  def sc_kernel(feat_hbm, src_hbm, dst_hbm, out_hbm,
                src_v, dst_v, rows_v, acc, gsem, ssem):
    c = lax.axis_index("c")
    s = lax.axis_index("s")

    zero16 = jnp.zeros((16,), jnp.float32)

    def _zrow(i, carry):
      for t in range(DH // 16):
        rows_v[0, i, pl.ds(t * 16, 16)] = zero16
      return carry

    lax.fori_loop(0, CHUNK, _zrow, 0)
    row0 = s * ROWS_PER_TILE
    for b in range(ROWS_PER_TILE // CHUNK):
      pltpu.sync_copy(rows_v.at[0], acc.at[pl.ds(row0 + b * CHUNK, CHUNK)])
    plsc.subcore_barrier()

    pltpu.sync_copy(src_hbm.at[c, s], src_v)
    pltpu.sync_copy(dst_hbm.at[s], dst_v)

    def _start_gather(j, b):
      pltpu.async_copy(feat_hbm.at[src_v.at[j]], rows_v.at[b], gsem.at[b])

    def _wait_gather(j, b):
      pltpu.make_async_copy(
          feat_hbm.at[src_v.at[j]], rows_v.at[b], gsem.at[b]).wait()

    def _start_scatter(j, b):
      pltpu.async_copy(rows_v.at[b], acc.at[dst_v.at[j]], ssem.at[b], add=True)

    def _wait_scatter(j, b):
      pltpu.make_async_copy(
          rows_v.at[b], acc.at[dst_v.at[j]], ssem.at[b]).wait()

    for b in range(NBUF):
      _start_gather(b, b)

    def _group(g, carry):
      for b in range(NBUF):
        j = g * NBUF + b
        _wait_gather(j, b)
        _start_scatter(j, b)
        jj = j - LAG
        bb = (b - LAG) % NBUF

        @pl.when(jj >= 0)
        def _():
          _wait_scatter(jj, bb)
          jn = jj + NBUF

          @pl.when(jn < NCHUNK)
          def _():
            _start_gather(jn, bb)

      return carry

    lax.fori_loop(0, NCHUNK // NBUF, _group, 0)
    for t in range(LAG):
      jj = NCHUNK - LAG + t
      _wait_scatter(jj, jj % NBUF)
    plsc.subcore_barrier()

    pltpu.sync_copy(
        acc.at[pl.ds(row0, ROWS_PER_TILE)],
        out_hbm.at[pl.ds(c * N_PAD + row0, ROWS_PER_TILE)])

  return sc_kernel


_sc_call = _make_sc_call()


def kernel(feat, edge_index):
  ei = edge_index.astype(jnp.int32)
  npad = E_PAD - N_EDGES
  src = jnp.concatenate([ei[0], jnp.zeros((npad,), jnp.int32)])
  dst = jnp.concatenate([ei[1], jnp.full((npad,), TRASH_ROW, jnp.int32)])
  src2 = jnp.stack([src, src + N_NODES]).reshape(2, 16, NCHUNK, CHUNK)
  dst2 = dst.reshape(16, NCHUNK, CHUNK)
  feat2 = feat.reshape(N_NODES, 2, DH).transpose(1, 0, 2).reshape(2 * N_NODES, DH)
  out2 = _sc_call(feat2, src2, dst2)
  out2 = out2.reshape(2, N_PAD, DH)[:, :N_NODES, :]
  return out2.transpose(1, 0, 2).reshape(N_NODES, D_FEAT)
